# Initial kernel scaffold; baseline (speedup 1.0000x reference)
import jax
import jax.numpy as jnp
from jax.experimental import pallas as pl


def kernel(developer, publisher, genres, tags, numeric_feats, dev_table, pub_table, gen_table, tag_table, num_W, num_b, W1, b1, W2, b2):
    raise NotImplementedError("write your pallas kernel here")


# speedup vs baseline: 2.8053x; 2.8053x over previous
import functools

import jax
import jax.numpy as jnp
from jax import lax
from jax.experimental import pallas as pl
from jax.experimental.pallas import tpu as pltpu
from jax.experimental.pallas import tpu_sc as plsc

B = 16384
EMB = 32
NC, NS = 2, 16
NW = NC * NS
BPW = B // NW
TAG_K = 50
GEN_K = 20
TPC = 2
GPC = 4
TAG_ROW = TPC * TAG_K + 4
GEN_ROW = GPC * GEN_K
TAG_COPIES = BPW // TPC
GEN_COPIES = BPW // GPC
DP_COPIES = BPW // 128


def _pool_rows(buf, k, n_items, out_buf, out_base, pad_rows):
    for it in range(n_items):
        r0 = it * k
        acc0 = buf[r0, pl.ds(0, 16)]
        acc1 = buf[r0, pl.ds(16, 16)]
        for j in range(1, k):
            acc0 = acc0 + buf[r0 + j, pl.ds(0, 16)]
            acc1 = acc1 + buf[r0 + j, pl.ds(16, 16)]
        out_buf[out_base + it, pl.ds(0, 16)] = acc0
        out_buf[out_base + it, pl.ds(16, 16)] = acc1


def _sc_gather_pool(dev_idx2, pub_idx2, tag_idx2, gen_idx2,
                    dev_table, pub_table, gen_table, tag_table):
    mesh = plsc.VectorSubcoreMesh(core_axis_name="c", subcore_axis_name="s")
    out_sds = jax.ShapeDtypeStruct((B, EMB), jnp.float32)

    @functools.partial(
        pl.kernel, mesh=mesh,
        out_type=[out_sds, out_sds, out_sds, out_sds],
        compiler_params=pltpu.CompilerParams(use_tc_tiling_on_sc=False),
        scratch_types=[
            pltpu.VMEM((TAG_COPIES, TAG_ROW), jnp.int32),
            pltpu.VMEM((GEN_COPIES, GEN_ROW), jnp.int32),
            pltpu.VMEM((DP_COPIES, 128), jnp.int32),
            pltpu.VMEM((DP_COPIES, 128), jnp.int32),
            pltpu.VMEM((TAG_ROW, EMB), jnp.float32),
            pltpu.VMEM((TAG_ROW, EMB), jnp.float32),
            pltpu.VMEM((GEN_ROW, EMB), jnp.float32),
            pltpu.VMEM((GEN_ROW, EMB), jnp.float32),
            pltpu.VMEM((128, EMB), jnp.float32),
            pltpu.VMEM((128, EMB), jnp.float32),
            pltpu.VMEM((BPW, EMB), jnp.float32),
            pltpu.SemaphoreType.DMA,
            pltpu.SemaphoreType.DMA,
            pltpu.SemaphoreType.DMA,
            pltpu.SemaphoreType.DMA,
            pltpu.SemaphoreType.DMA,
        ],
    )
    def k(dev_i_hbm, pub_i_hbm, tag_i_hbm, gen_i_hbm,
          devT, pubT, tagT, genT,
          out_dev, out_pub, out_tag, out_gen,
          tag_idx, gen_idx, dev_idx, pub_idx,
          tag0, tag1, gen0, gen1, dp0, dp1, acc_buf,
          sem_dp, sem_t0, sem_t1, sem_g0, sem_g1):
        wid = lax.axis_index("s") * NC + lax.axis_index("c")
        base = wid * BPW

        pltpu.sync_copy(tag_i_hbm.at[pl.ds(wid * TAG_COPIES, TAG_COPIES)], tag_idx)
        pltpu.sync_copy(gen_i_hbm.at[pl.ds(wid * GEN_COPIES, GEN_COPIES)], gen_idx)
        pltpu.sync_copy(dev_i_hbm.at[pl.ds(wid * DP_COPIES, DP_COPIES)], dev_idx)
        pltpu.sync_copy(pub_i_hbm.at[pl.ds(wid * DP_COPIES, DP_COPIES)], pub_idx)

        for r in range(DP_COPIES):
            pltpu.async_copy(devT.at[dev_idx.at[r]], dp0, sem_dp).wait()
            pltpu.sync_copy(dp0, out_dev.at[pl.ds(base + r * 128, 128)])
            pltpu.async_copy(pubT.at[pub_idx.at[r]], dp1, sem_dp).wait()
            pltpu.sync_copy(dp1, out_pub.at[pl.ds(base + r * 128, 128)])

        pltpu.async_copy(tagT.at[tag_idx.at[0]], tag0, sem_t0)

        @pl.loop(0, TAG_COPIES, step=2)
        def _(p):
            pltpu.async_copy(tagT.at[tag_idx.at[p + 1]], tag1, sem_t1)
            pltpu.make_async_copy(tagT.at[tag_idx.at[p]], tag0, sem_t0).wait()
            _pool_rows(tag0, TAG_K, TPC, acc_buf, p * TPC, 4)

            @pl.when(p + 2 < TAG_COPIES)
            def _():
                pltpu.async_copy(tagT.at[tag_idx.at[p + 2]], tag0, sem_t0)

            pltpu.make_async_copy(tagT.at[tag_idx.at[p + 1]], tag1, sem_t1).wait()
            _pool_rows(tag1, TAG_K, TPC, acc_buf, (p + 1) * TPC, 4)

        pltpu.sync_copy(acc_buf, out_tag.at[pl.ds(base, BPW)])

        pltpu.async_copy(genT.at[gen_idx.at[0]], gen0, sem_g0)

        @pl.loop(0, GEN_COPIES, step=2)
        def _(p):
            pltpu.async_copy(genT.at[gen_idx.at[p + 1]], gen1, sem_g1)
            pltpu.make_async_copy(genT.at[gen_idx.at[p]], gen0, sem_g0).wait()
            _pool_rows(gen0, GEN_K, GPC, acc_buf, p * GPC, 0)

            @pl.when(p + 2 < GEN_COPIES)
            def _():
                pltpu.async_copy(genT.at[gen_idx.at[p + 2]], gen0, sem_g0)

            pltpu.make_async_copy(genT.at[gen_idx.at[p + 1]], gen1, sem_g1).wait()
            _pool_rows(gen1, GEN_K, GPC, acc_buf, (p + 1) * GPC, 0)

        pltpu.sync_copy(acc_buf, out_gen.at[pl.ds(base, BPW)])

    return k(dev_idx2, pub_idx2, tag_idx2, gen_idx2,
             dev_table, pub_table, tag_table, gen_table)


BM = 1024


def _mlp_body(dev_ref, pub_ref, gen_ref, tag_ref, nf_ref,
              numW_ref, numb_ref, W1_ref, b1_ref, W2_ref, b2_ref, out_ref):
    f32 = jnp.float32
    W1 = W1_ref[...]
    dev = dev_ref[...] * 1.5
    pub = pub_ref[...] * 1.5
    gen = gen_ref[...] * (1.0 / GEN_K)
    tag = tag_ref[...] * (1.0 / TAG_K)
    nf = nf_ref[...]
    num = (nf[:, 0:1] * numW_ref[0:1, :] + nf[:, 1:2] * numW_ref[1:2, :]
           + numb_ref[...])
    acc = jnp.dot(dev, W1[0:32], preferred_element_type=f32)
    acc = acc + jnp.dot(pub, W1[32:64], preferred_element_type=f32)
    acc = acc + jnp.dot(gen, W1[64:96], preferred_element_type=f32)
    acc = acc + jnp.dot(tag, W1[96:128], preferred_element_type=f32)
    acc = acc + jnp.dot(num, W1[128:160], preferred_element_type=f32)
    h = jnp.maximum(acc + b1_ref[...], 0.0)
    out_ref[...] = jnp.dot(h, W2_ref[...], preferred_element_type=f32) + b2_ref[...]


def _mlp(dev_emb, pub_emb, gen_sum, tag_sum, numeric_feats,
         num_W, num_b, W1, b1, W2, b2):
    grid = (B // BM,)
    full = lambda i: (0, 0)
    row = lambda i: (i, 0)
    return pl.pallas_call(
        _mlp_body,
        grid=grid,
        in_specs=[
            pl.BlockSpec((BM, EMB), row),
            pl.BlockSpec((BM, EMB), row),
            pl.BlockSpec((BM, EMB), row),
            pl.BlockSpec((BM, EMB), row),
            pl.BlockSpec((BM, 2), row),
            pl.BlockSpec((2, EMB), full),
            pl.BlockSpec((1, EMB), full),
            pl.BlockSpec((5 * EMB, 128), full),
            pl.BlockSpec((1, 128), full),
            pl.BlockSpec((128, 64), full),
            pl.BlockSpec((1, 64), full),
        ],
        out_specs=pl.BlockSpec((BM, 64), row),
        out_shape=jax.ShapeDtypeStruct((B, 64), jnp.float32),
    )(dev_emb, pub_emb, gen_sum, tag_sum, numeric_feats,
      num_W, num_b.reshape(1, EMB), W1, b1.reshape(1, 128),
      W2, b2.reshape(1, 64))


def kernel(developer, publisher, genres, tags, numeric_feats,
           dev_table, pub_table, gen_table, tag_table,
           num_W, num_b, W1, b1, W2, b2):
    developer = developer.astype(jnp.int32)
    publisher = publisher.astype(jnp.int32)
    genres = genres.astype(jnp.int32)
    tags = tags.astype(jnp.int32)

    dev_idx2 = developer.reshape(B // 128, 128)
    pub_idx2 = publisher.reshape(B // 128, 128)
    tag_idx2 = jnp.pad(tags.reshape(B // TPC, TPC * TAG_K),
                       ((0, 0), (0, TAG_ROW - TPC * TAG_K)))
    gen_idx2 = genres.reshape(B // GPC, GPC * GEN_K)

    dev_emb, pub_emb, tag_sum, gen_sum = _sc_gather_pool(
        dev_idx2, pub_idx2, tag_idx2, gen_idx2,
        dev_table, pub_table, gen_table, tag_table)

    return _mlp(dev_emb, pub_emb, gen_sum, tag_sum, numeric_feats,
                num_W, num_b, W1, b1, W2, b2)


# speedup vs baseline: 4.2827x; 1.5267x over previous
import functools

import jax
import jax.numpy as jnp
from jax import lax
from jax.experimental import pallas as pl
from jax.experimental.pallas import tpu as pltpu
from jax.experimental.pallas import tpu_sc as plsc

B = 16384
EMB = 32
NC, NS = 2, 16
NW = NC * NS
BPW = B // NW
TAG_K = 50
GEN_K = 20
TPG = 4
GPG = 8
TAG_COPIES = BPW // TPG
GEN_COPIES = BPW // GPG
DP_COPIES = BPW // 128
RING = 4


def _pool_copy(buf, k, n_items, out_buf, out_base):
    @pl.loop(0, n_items)
    def _(it):
        r0 = it * k
        acc0 = buf[r0, pl.ds(0, 16)]
        acc1 = buf[r0, pl.ds(16, 16)]
        for j in range(1, k):
            acc0 = acc0 + buf[r0 + j, pl.ds(0, 16)]
            acc1 = acc1 + buf[r0 + j, pl.ds(16, 16)]
        out_buf[out_base + it, pl.ds(0, 16)] = acc0
        out_buf[out_base + it, pl.ds(16, 16)] = acc1


def _sc_gather_pool(developer, publisher, tags, genres,
                    dev_table, pub_table, gen_table, tag_table):
    mesh = plsc.VectorSubcoreMesh(core_axis_name="c", subcore_axis_name="s")

    @functools.partial(
        pl.kernel, mesh=mesh,
        out_type=jax.ShapeDtypeStruct((B, 4 * EMB), jnp.float32),
        compiler_params=pltpu.CompilerParams(use_tc_tiling_on_sc=False),
        scratch_types=[
            pltpu.VMEM((BPW * TAG_K,), jnp.int32),
            pltpu.VMEM((BPW * GEN_K,), jnp.int32),
            pltpu.VMEM((BPW,), jnp.int32),
            pltpu.VMEM((BPW,), jnp.int32),
            pltpu.VMEM((RING, TPG * TAG_K, EMB), jnp.float32),
            pltpu.VMEM((RING, GPG * GEN_K, EMB), jnp.float32),
            pltpu.VMEM((128, EMB), jnp.float32),
            pltpu.VMEM((128, EMB), jnp.float32),
            pltpu.VMEM((BPW, EMB), jnp.float32),
            pltpu.VMEM((BPW, EMB), jnp.float32),
            pltpu.SemaphoreType.DMA((RING,)),
            pltpu.SemaphoreType.DMA((RING,)),
            pltpu.SemaphoreType.DMA,
        ],
    )
    def k(dev_i_hbm, pub_i_hbm, tag_i_hbm, gen_i_hbm,
          devT, pubT, tagT, genT, out,
          tag_idx, gen_idx, dev_idx, pub_idx,
          tag_bufs, gen_bufs, dp0, dp1, acc_a, acc_b,
          sem_t, sem_g, sem_dp):
        wid = lax.axis_index("s") * NC + lax.axis_index("c")
        base = wid * BPW

        pltpu.sync_copy(tag_i_hbm.at[pl.ds(base * TAG_K, BPW * TAG_K)], tag_idx)
        pltpu.sync_copy(gen_i_hbm.at[pl.ds(base * GEN_K, BPW * GEN_K)], gen_idx)
        pltpu.sync_copy(dev_i_hbm.at[pl.ds(base, BPW)], dev_idx)
        pltpu.sync_copy(pub_i_hbm.at[pl.ds(base, BPW)], pub_idx)

        for r in range(DP_COPIES):
            pltpu.async_copy(devT.at[dev_idx.at[pl.ds(r * 128, 128)]], dp0,
                             sem_dp).wait()
            pltpu.sync_copy(dp0, out.at[pl.ds(base + r * 128, 128),
                                        pl.ds(0, EMB)])
            pltpu.async_copy(pubT.at[pub_idx.at[pl.ds(r * 128, 128)]], dp1,
                             sem_dp).wait()
            pltpu.sync_copy(dp1, out.at[pl.ds(base + r * 128, 128),
                                        pl.ds(EMB, EMB)])

        for b in range(RING):
            pltpu.async_copy(tagT.at[tag_idx.at[pl.ds(b * TPG * TAG_K, TPG * TAG_K)]],
                             tag_bufs.at[b], sem_t.at[b])

        @pl.loop(0, TAG_COPIES, step=RING)
        def _(p0):
            for b in range(RING):
                p = p0 + b
                pltpu.make_async_copy(
                    tagT.at[tag_idx.at[pl.ds(p * TPG * TAG_K, TPG * TAG_K)]],
                    tag_bufs.at[b], sem_t.at[b]).wait()
                _pool_copy(tag_bufs.at[b], TAG_K, TPG, acc_a, p * TPG)

                @pl.when(p + RING < TAG_COPIES)
                def _():
                    pltpu.async_copy(
                        tagT.at[tag_idx.at[pl.ds((p + RING) * TPG * TAG_K, TPG * TAG_K)]],
                        tag_bufs.at[b], sem_t.at[b])

        pltpu.sync_copy(acc_a, out.at[pl.ds(base, BPW), pl.ds(3 * EMB, EMB)])

        for b in range(RING):
            pltpu.async_copy(genT.at[gen_idx.at[pl.ds(b * GPG * GEN_K, GPG * GEN_K)]],
                             gen_bufs.at[b], sem_g.at[b])

        @pl.loop(0, GEN_COPIES, step=RING)
        def _(p0):
            for b in range(RING):
                p = p0 + b
                pltpu.make_async_copy(
                    genT.at[gen_idx.at[pl.ds(p * GPG * GEN_K, GPG * GEN_K)]],
                    gen_bufs.at[b], sem_g.at[b]).wait()
                _pool_copy(gen_bufs.at[b], GEN_K, GPG, acc_b, p * GPG)

                @pl.when(p + RING < GEN_COPIES)
                def _():
                    pltpu.async_copy(
                        genT.at[gen_idx.at[pl.ds((p + RING) * GPG * GEN_K, GPG * GEN_K)]],
                        gen_bufs.at[b], sem_g.at[b])

        pltpu.sync_copy(acc_b, out.at[pl.ds(base, BPW), pl.ds(2 * EMB, EMB)])

    return k(developer, publisher, tags, genres,
             dev_table, pub_table, tag_table, gen_table)


BM = 1024


def _mlp_body(emb_ref, nf_ref, numW_ref, numb_ref, W1_ref, b1_ref,
              W2_ref, b2_ref, out_ref):
    f32 = jnp.float32
    lanes = lax.broadcasted_iota(jnp.int32, (1, 4 * EMB), 1)
    scale = jnp.where(lanes < 2 * EMB, 1.5,
                      jnp.where(lanes < 3 * EMB, 1.0 / GEN_K, 1.0 / TAG_K))
    emb = emb_ref[...] * scale
    nf = nf_ref[...]
    num = (nf[:, 0:1] * numW_ref[0:1, :] + nf[:, 1:2] * numW_ref[1:2, :]
           + numb_ref[...])
    W1 = W1_ref[...]
    acc = jnp.dot(emb, W1[0:128], preferred_element_type=f32)
    acc = acc + jnp.dot(num, W1[128:160], preferred_element_type=f32)
    h = jnp.maximum(acc + b1_ref[...], 0.0)
    out_ref[...] = jnp.dot(h, W2_ref[...], preferred_element_type=f32) + b2_ref[...]


def _mlp(emb, numeric_feats, num_W, num_b, W1, b1, W2, b2):
    grid = (B // BM,)
    full = lambda i: (0, 0)
    row = lambda i: (i, 0)
    return pl.pallas_call(
        _mlp_body,
        grid=grid,
        in_specs=[
            pl.BlockSpec((BM, 4 * EMB), row),
            pl.BlockSpec((BM, 2), row),
            pl.BlockSpec((2, EMB), full),
            pl.BlockSpec((1, EMB), full),
            pl.BlockSpec((5 * EMB, 128), full),
            pl.BlockSpec((1, 128), full),
            pl.BlockSpec((128, 64), full),
            pl.BlockSpec((1, 64), full),
        ],
        out_specs=pl.BlockSpec((BM, 64), row),
        out_shape=jax.ShapeDtypeStruct((B, 64), jnp.float32),
    )(emb, numeric_feats, num_W, num_b.reshape(1, EMB),
      W1, b1.reshape(1, 128), W2, b2.reshape(1, 64))


def kernel(developer, publisher, genres, tags, numeric_feats,
           dev_table, pub_table, gen_table, tag_table,
           num_W, num_b, W1, b1, W2, b2):
    developer = developer.astype(jnp.int32)
    publisher = publisher.astype(jnp.int32)
    genres = genres.astype(jnp.int32)
    tags = tags.astype(jnp.int32)

    emb = _sc_gather_pool(developer, publisher,
                          tags.reshape(B * TAG_K), genres.reshape(B * GEN_K),
                          dev_table, pub_table, gen_table, tag_table)

    return _mlp(emb, numeric_feats, num_W, num_b, W1, b1, W2, b2)


# speedup vs baseline: 6.0755x; 1.4186x over previous
import functools

import jax
import jax.numpy as jnp
from jax import lax
from jax.experimental import pallas as pl
from jax.experimental.pallas import tpu as pltpu
from jax.experimental.pallas import tpu_sc as plsc

B = 16384
EMB = 32
NC, NS = 2, 16
NW = NC * NS
BPW = B // NW
TAG_K = 50
GEN_K = 20
TPG = 4
GPG = 8
TAG_COPIES = BPW // TPG
GEN_COPIES = BPW // GPG
DP_COPIES = BPW // 128
RING = 4


def _pool_copy(buf, k, n_items, out_buf, out_base):
    @pl.loop(0, n_items)
    def _(it):
        r0 = it * k
        acc0 = buf[r0, pl.ds(0, 16)]
        acc1 = buf[r0, pl.ds(16, 16)]
        for j in range(1, k):
            acc0 = acc0 + buf[r0 + j, pl.ds(0, 16)]
            acc1 = acc1 + buf[r0 + j, pl.ds(16, 16)]
        out_buf[out_base + it, pl.ds(0, 16)] = acc0
        out_buf[out_base + it, pl.ds(16, 16)] = acc1


def _sc_gather_pool(developer, publisher, tags, genres,
                    dev_table, pub_table, gen_table, tag_table):
    mesh = plsc.VectorSubcoreMesh(core_axis_name="c", subcore_axis_name="s")

    @functools.partial(
        pl.kernel, mesh=mesh,
        out_type=jax.ShapeDtypeStruct((B, 4 * EMB), jnp.float32),
        compiler_params=pltpu.CompilerParams(use_tc_tiling_on_sc=False),
        scratch_types=[
            pltpu.VMEM((BPW * TAG_K,), jnp.int32),
            pltpu.VMEM((BPW * GEN_K,), jnp.int32),
            pltpu.VMEM((BPW,), jnp.int32),
            pltpu.VMEM((BPW,), jnp.int32),
            pltpu.VMEM((RING, TPG * TAG_K, EMB), jnp.float32),
            pltpu.VMEM((RING, GPG * GEN_K, EMB), jnp.float32),
            pltpu.VMEM((128, EMB), jnp.float32),
            pltpu.VMEM((128, EMB), jnp.float32),
            pltpu.VMEM((BPW, EMB), jnp.float32),
            pltpu.VMEM((BPW, EMB), jnp.float32),
            pltpu.SemaphoreType.DMA((RING,)),
            pltpu.SemaphoreType.DMA((RING,)),
            pltpu.SemaphoreType.DMA,
        ],
    )
    def k(dev_i_hbm, pub_i_hbm, tag_i_hbm, gen_i_hbm,
          devT, pubT, tagT, genT, out,
          tag_idx, gen_idx, dev_idx, pub_idx,
          tag_bufs, gen_bufs, dp0, dp1, acc_a, acc_b,
          sem_t, sem_g, sem_dp):
        wid = lax.axis_index("s") * NC + lax.axis_index("c")
        base = wid * BPW

        pltpu.sync_copy(tag_i_hbm.at[pl.ds(base * TAG_K, BPW * TAG_K)], tag_idx)
        pltpu.sync_copy(gen_i_hbm.at[pl.ds(base * GEN_K, BPW * GEN_K)], gen_idx)
        pltpu.sync_copy(dev_i_hbm.at[pl.ds(base, BPW)], dev_idx)
        pltpu.sync_copy(pub_i_hbm.at[pl.ds(base, BPW)], pub_idx)

        for r in range(DP_COPIES):
            pltpu.async_copy(devT.at[dev_idx.at[pl.ds(r * 128, 128)]], dp0,
                             sem_dp).wait()
            pltpu.sync_copy(dp0, out.at[pl.ds(base + r * 128, 128),
                                        pl.ds(0, EMB)])
            pltpu.async_copy(pubT.at[pub_idx.at[pl.ds(r * 128, 128)]], dp1,
                             sem_dp).wait()
            pltpu.sync_copy(dp1, out.at[pl.ds(base + r * 128, 128),
                                        pl.ds(EMB, EMB)])

        for b in range(RING):
            pltpu.async_copy(tagT.at[tag_idx.at[pl.ds(b * TPG * TAG_K, TPG * TAG_K)]],
                             tag_bufs.at[b], sem_t.at[b])

        @pl.loop(0, TAG_COPIES, step=RING)
        def _(p0):
            for b in range(RING):
                p = p0 + b
                pltpu.make_async_copy(
                    tagT.at[tag_idx.at[pl.ds(p * TPG * TAG_K, TPG * TAG_K)]],
                    tag_bufs.at[b], sem_t.at[b]).wait()
                _pool_copy(tag_bufs.at[b], TAG_K, TPG, acc_a, p * TPG)

                @pl.when(p + RING < TAG_COPIES)
                def _():
                    pltpu.async_copy(
                        tagT.at[tag_idx.at[pl.ds((p + RING) * TPG * TAG_K, TPG * TAG_K)]],
                        tag_bufs.at[b], sem_t.at[b])

        pltpu.sync_copy(acc_a, out.at[pl.ds(base, BPW), pl.ds(3 * EMB, EMB)])

        for b in range(RING):
            pltpu.async_copy(genT.at[gen_idx.at[pl.ds(b * GPG * GEN_K, GPG * GEN_K)]],
                             gen_bufs.at[b], sem_g.at[b])

        @pl.loop(0, GEN_COPIES, step=RING)
        def _(p0):
            for b in range(RING):
                p = p0 + b
                pltpu.make_async_copy(
                    genT.at[gen_idx.at[pl.ds(p * GPG * GEN_K, GPG * GEN_K)]],
                    gen_bufs.at[b], sem_g.at[b]).wait()
                _pool_copy(gen_bufs.at[b], GEN_K, GPG, acc_b, p * GPG)

                @pl.when(p + RING < GEN_COPIES)
                def _():
                    pltpu.async_copy(
                        genT.at[gen_idx.at[pl.ds((p + RING) * GPG * GEN_K, GPG * GEN_K)]],
                        gen_bufs.at[b], sem_g.at[b])

        pltpu.sync_copy(acc_b, out.at[pl.ds(base, BPW), pl.ds(2 * EMB, EMB)])

    return k(developer, publisher, tags, genres,
             dev_table, pub_table, tag_table, gen_table)


def _regroup_body(a_ref, b_ref, c_ref, d_ref, out_ref):
    out_ref[...] = jnp.concatenate(
        [a_ref[...].T, b_ref[...].T, c_ref[...].T, d_ref[...].T], axis=1)


def _to_banked(table, bk):
    t = table.T
    v = t.shape[1]
    nb = (((v + 3) // 4) + bk - 1) // bk
    qs = nb * bk
    total_blocks = (v + bk - 1) // bk
    b3 = total_blocks - nb
    assert b3 >= 0 and b3 * bk <= 3 * qs
    starts = [0, nb, 2 * nb, b3]
    specs = [
        pl.BlockSpec((32, bk), (lambda s: (lambda i: (0, i + s)))(s))
        for s in starts
    ]
    out = pl.pallas_call(
        _regroup_body,
        grid=(nb,),
        in_specs=specs,
        out_specs=pl.BlockSpec((bk, 128), lambda i: (i, 0)),
        out_shape=jax.ShapeDtypeStruct((qs, 128), jnp.float32),
    )(t, t, t, t)
    return out.reshape(4 * qs, 32), (qs, b3 * bk)


def _bank_idx(idx, qinfo):
    qs, s3 = qinfo
    k = jnp.minimum(idx // qs, 3)
    base = jnp.where(k == 3, s3, k * qs)
    return (idx - base) * 4 + k


BM = 1024


def _mlp_body(emb_ref, nf_ref, numW_ref, numb_ref, W1_ref, b1_ref,
              W2_ref, b2_ref, out_ref):
    f32 = jnp.float32
    lanes = lax.broadcasted_iota(jnp.int32, (1, 4 * EMB), 1)
    scale = jnp.where(lanes < 2 * EMB, 1.5,
                      jnp.where(lanes < 3 * EMB, 1.0 / GEN_K, 1.0 / TAG_K))
    emb = emb_ref[...] * scale
    nf = nf_ref[...]
    num = (nf[:, 0:1] * numW_ref[0:1, :] + nf[:, 1:2] * numW_ref[1:2, :]
           + numb_ref[...])
    W1 = W1_ref[...]
    acc = jnp.dot(emb, W1[0:128], preferred_element_type=f32)
    acc = acc + jnp.dot(num, W1[128:160], preferred_element_type=f32)
    h = jnp.maximum(acc + b1_ref[...], 0.0)
    out_ref[...] = jnp.dot(h, W2_ref[...], preferred_element_type=f32) + b2_ref[...]


def _mlp(emb, numeric_feats, num_W, num_b, W1, b1, W2, b2):
    grid = (B // BM,)
    full = lambda i: (0, 0)
    row = lambda i: (i, 0)
    return pl.pallas_call(
        _mlp_body,
        grid=grid,
        in_specs=[
            pl.BlockSpec((BM, 4 * EMB), row),
            pl.BlockSpec((BM, 2), row),
            pl.BlockSpec((2, EMB), full),
            pl.BlockSpec((1, EMB), full),
            pl.BlockSpec((5 * EMB, 128), full),
            pl.BlockSpec((1, 128), full),
            pl.BlockSpec((128, 64), full),
            pl.BlockSpec((1, 64), full),
        ],
        out_specs=pl.BlockSpec((BM, 64), row),
        out_shape=jax.ShapeDtypeStruct((B, 64), jnp.float32),
    )(emb, numeric_feats, num_W, num_b.reshape(1, EMB),
      W1, b1.reshape(1, 128), W2, b2.reshape(1, 64))


def kernel(developer, publisher, genres, tags, numeric_feats,
           dev_table, pub_table, gen_table, tag_table,
           num_W, num_b, W1, b1, W2, b2):
    developer = developer.astype(jnp.int32)
    publisher = publisher.astype(jnp.int32)
    genres = genres.astype(jnp.int32)
    tags = tags.astype(jnp.int32)

    dev_lin, qi_dp = _to_banked(dev_table, 4096)
    pub_lin, _ = _to_banked(pub_table, 4096)
    gen_lin, qi_gen = _to_banked(gen_table, 128)
    tag_lin, qi_tag = _to_banked(tag_table, 4096)

    emb = _sc_gather_pool(_bank_idx(developer, qi_dp),
                          _bank_idx(publisher, qi_dp),
                          _bank_idx(tags, qi_tag).reshape(B * TAG_K),
                          _bank_idx(genres, qi_gen).reshape(B * GEN_K),
                          dev_lin, pub_lin, gen_lin, tag_lin)

    return _mlp(emb, numeric_feats, num_W, num_b, W1, b1, W2, b2)


# speedup vs baseline: 9.6111x; 1.5819x over previous
import functools

import jax
import jax.numpy as jnp
from jax import lax
from jax.experimental import pallas as pl
from jax.experimental.pallas import tpu as pltpu
from jax.experimental.pallas import tpu_sc as plsc

B = 16384
EMB = 32
NC, NS = 2, 16
NW = NC * NS
BPW = B // NW
TAG_K = 50
GEN_K = 20
TPG = 4
GPG = 8
TAG_COPIES = BPW // TPG
GEN_COPIES = BPW // GPG
DP_COPIES = BPW // 128
RING = 4


def _pool_copy(buf, k, n_items, out_buf, out_base):
    @pl.loop(0, n_items)
    def _(it):
        r0 = it * k
        acc0 = buf[r0, pl.ds(0, 16)]
        acc1 = buf[r0, pl.ds(16, 16)]
        for j in range(1, k):
            acc0 = acc0 + buf[r0 + j, pl.ds(0, 16)]
            acc1 = acc1 + buf[r0 + j, pl.ds(16, 16)]
        out_buf[out_base + it, pl.ds(0, 16)] = acc0
        out_buf[out_base + it, pl.ds(16, 16)] = acc1


def _sc_gather_pool(developer, publisher, tags, genres,
                    dev_table, pub_table, gen_table, tag_table):
    mesh = plsc.VectorSubcoreMesh(core_axis_name="c", subcore_axis_name="s")

    @functools.partial(
        pl.kernel, mesh=mesh,
        out_type=jax.ShapeDtypeStruct((B, 4 * EMB), jnp.float32),
        compiler_params=pltpu.CompilerParams(use_tc_tiling_on_sc=False),
        scratch_types=[
            pltpu.VMEM((BPW * TAG_K,), jnp.int32),
            pltpu.VMEM((BPW * GEN_K,), jnp.int32),
            pltpu.VMEM((BPW,), jnp.int32),
            pltpu.VMEM((BPW,), jnp.int32),
            pltpu.VMEM((RING, TPG * TAG_K, EMB), jnp.float32),
            pltpu.VMEM((RING, GPG * GEN_K, EMB), jnp.float32),
            pltpu.VMEM((128, EMB), jnp.float32),
            pltpu.VMEM((128, EMB), jnp.float32),
            pltpu.VMEM((BPW, EMB), jnp.float32),
            pltpu.VMEM((BPW, EMB), jnp.float32),
            pltpu.SemaphoreType.DMA((RING,)),
            pltpu.SemaphoreType.DMA((RING,)),
            pltpu.SemaphoreType.DMA,
        ],
    )
    def k(dev_i_hbm, pub_i_hbm, tag_i_hbm, gen_i_hbm,
          devT, pubT, tagT, genT, out,
          tag_idx, gen_idx, dev_idx, pub_idx,
          tag_bufs, gen_bufs, dp0, dp1, acc_a, acc_b,
          sem_t, sem_g, sem_dp):
        wid = lax.axis_index("s") * NC + lax.axis_index("c")
        base = wid * BPW

        pltpu.sync_copy(tag_i_hbm.at[pl.ds(base * TAG_K, BPW * TAG_K)], tag_idx)
        pltpu.sync_copy(gen_i_hbm.at[pl.ds(base * GEN_K, BPW * GEN_K)], gen_idx)
        pltpu.sync_copy(dev_i_hbm.at[pl.ds(base, BPW)], dev_idx)
        pltpu.sync_copy(pub_i_hbm.at[pl.ds(base, BPW)], pub_idx)

        for r in range(DP_COPIES):
            pltpu.async_copy(devT.at[dev_idx.at[pl.ds(r * 128, 128)]], dp0,
                             sem_dp).wait()
            pltpu.sync_copy(dp0, out.at[pl.ds(base + r * 128, 128),
                                        pl.ds(0, EMB)])
            pltpu.async_copy(pubT.at[pub_idx.at[pl.ds(r * 128, 128)]], dp1,
                             sem_dp).wait()
            pltpu.sync_copy(dp1, out.at[pl.ds(base + r * 128, 128),
                                        pl.ds(EMB, EMB)])

        for b in range(RING):
            pltpu.async_copy(tagT.at[tag_idx.at[pl.ds(b * TPG * TAG_K, TPG * TAG_K)]],
                             tag_bufs.at[b], sem_t.at[b])

        @pl.loop(0, TAG_COPIES, step=RING)
        def _(p0):
            for b in range(RING):
                p = p0 + b
                pltpu.make_async_copy(
                    tagT.at[tag_idx.at[pl.ds(p * TPG * TAG_K, TPG * TAG_K)]],
                    tag_bufs.at[b], sem_t.at[b]).wait()
                _pool_copy(tag_bufs.at[b], TAG_K, TPG, acc_a, p * TPG)

                @pl.when(p + RING < TAG_COPIES)
                def _():
                    pltpu.async_copy(
                        tagT.at[tag_idx.at[pl.ds((p + RING) * TPG * TAG_K, TPG * TAG_K)]],
                        tag_bufs.at[b], sem_t.at[b])

        pltpu.sync_copy(acc_a, out.at[pl.ds(base, BPW), pl.ds(3 * EMB, EMB)])

        for b in range(RING):
            pltpu.async_copy(genT.at[gen_idx.at[pl.ds(b * GPG * GEN_K, GPG * GEN_K)]],
                             gen_bufs.at[b], sem_g.at[b])

        @pl.loop(0, GEN_COPIES, step=RING)
        def _(p0):
            for b in range(RING):
                p = p0 + b
                pltpu.make_async_copy(
                    genT.at[gen_idx.at[pl.ds(p * GPG * GEN_K, GPG * GEN_K)]],
                    gen_bufs.at[b], sem_g.at[b]).wait()
                _pool_copy(gen_bufs.at[b], GEN_K, GPG, acc_b, p * GPG)

                @pl.when(p + RING < GEN_COPIES)
                def _():
                    pltpu.async_copy(
                        genT.at[gen_idx.at[pl.ds((p + RING) * GPG * GEN_K, GPG * GEN_K)]],
                        gen_bufs.at[b], sem_g.at[b])

        pltpu.sync_copy(acc_b, out.at[pl.ds(base, BPW), pl.ds(2 * EMB, EMB)])

    return k(developer, publisher, tags, genres,
             dev_table, pub_table, tag_table, gen_table)


def _regroup_body(a_ref, b_ref, c_ref, d_ref, out_ref):
    cat = jnp.concatenate(
        [a_ref[...], b_ref[...], c_ref[...], d_ref[...]], axis=0)
    out_ref[...] = cat.T


def _to_banked(table, bk):
    t = table.T
    v = t.shape[1]
    nb = (((v + 3) // 4) + bk - 1) // bk
    qs = nb * bk
    total_blocks = (v + bk - 1) // bk
    b3 = total_blocks - nb
    assert b3 >= 0 and b3 * bk <= 3 * qs
    starts = [0, nb, 2 * nb, b3]
    specs = [
        pl.BlockSpec((32, bk), (lambda s: (lambda i: (0, i + s)))(s))
        for s in starts
    ]
    out = pl.pallas_call(
        _regroup_body,
        grid=(nb,),
        in_specs=specs,
        out_specs=pl.BlockSpec((bk, 128), lambda i: (i, 0)),
        out_shape=jax.ShapeDtypeStruct((qs, 128), jnp.float32),
    )(t, t, t, t)
    return out.reshape(4 * qs, 32), (qs, b3 * bk)


def _bank_idx(idx, qinfo):
    qs, s3 = qinfo
    k = jnp.minimum(idx // qs, 3)
    base = jnp.where(k == 3, s3, k * qs)
    return (idx - base) * 4 + k


BM = 1024


def _mlp_body(emb_ref, nf_ref, numW_ref, numb_ref, W1_ref, b1_ref,
              W2_ref, b2_ref, out_ref):
    f32 = jnp.float32
    lanes = lax.broadcasted_iota(jnp.int32, (1, 4 * EMB), 1)
    scale = jnp.where(lanes < 2 * EMB, 1.5,
                      jnp.where(lanes < 3 * EMB, 1.0 / GEN_K, 1.0 / TAG_K))
    emb = emb_ref[...] * scale
    nf = nf_ref[...]
    num = (nf[:, 0:1] * numW_ref[0:1, :] + nf[:, 1:2] * numW_ref[1:2, :]
           + numb_ref[...])
    W1 = W1_ref[...]
    acc = jnp.dot(emb, W1[0:128], preferred_element_type=f32)
    acc = acc + jnp.dot(num, W1[128:160], preferred_element_type=f32)
    h = jnp.maximum(acc + b1_ref[...], 0.0)
    out_ref[...] = jnp.dot(h, W2_ref[...], preferred_element_type=f32) + b2_ref[...]


def _mlp(emb, numeric_feats, num_W, num_b, W1, b1, W2, b2):
    grid = (B // BM,)
    full = lambda i: (0, 0)
    row = lambda i: (i, 0)
    return pl.pallas_call(
        _mlp_body,
        grid=grid,
        in_specs=[
            pl.BlockSpec((BM, 4 * EMB), row),
            pl.BlockSpec((BM, 2), row),
            pl.BlockSpec((2, EMB), full),
            pl.BlockSpec((1, EMB), full),
            pl.BlockSpec((5 * EMB, 128), full),
            pl.BlockSpec((1, 128), full),
            pl.BlockSpec((128, 64), full),
            pl.BlockSpec((1, 64), full),
        ],
        out_specs=pl.BlockSpec((BM, 64), row),
        out_shape=jax.ShapeDtypeStruct((B, 64), jnp.float32),
    )(emb, numeric_feats, num_W, num_b.reshape(1, EMB),
      W1, b1.reshape(1, 128), W2, b2.reshape(1, 64))


def kernel(developer, publisher, genres, tags, numeric_feats,
           dev_table, pub_table, gen_table, tag_table,
           num_W, num_b, W1, b1, W2, b2):
    developer = developer.astype(jnp.int32)
    publisher = publisher.astype(jnp.int32)
    genres = genres.astype(jnp.int32)
    tags = tags.astype(jnp.int32)

    dev_lin, qi_dp = _to_banked(dev_table, 4096)
    pub_lin, _ = _to_banked(pub_table, 4096)
    gen_lin, qi_gen = _to_banked(gen_table, 128)
    tag_lin, qi_tag = _to_banked(tag_table, 4096)

    emb = _sc_gather_pool(_bank_idx(developer, qi_dp),
                          _bank_idx(publisher, qi_dp),
                          _bank_idx(tags, qi_tag).reshape(B * TAG_K),
                          _bank_idx(genres, qi_gen).reshape(B * GEN_K),
                          dev_lin, pub_lin, gen_lin, tag_lin)

    return _mlp(emb, numeric_feats, num_W, num_b, W1, b1, W2, b2)
